# Initial kernel scaffold; baseline (speedup 1.0000x reference)
import jax
import jax.numpy as jnp
from jax.experimental import pallas as pl


def kernel(x_in, x_s, feature_msb, feature_lsb, weights):
    raise NotImplementedError("write your pallas kernel here")


# speedup vs baseline: 15.4676x; 15.4676x over previous
import functools

import jax
import jax.numpy as jnp
from jax import lax
from jax.experimental import pallas as pl
from jax.experimental.pallas import tpu as pltpu
from jax.experimental.pallas import tpu_sc as plsc

H = 512
W = 512
N = H * W
D = 20
KC = 17 * 17 * 17
KP = 4992
NC = 2
NS = 16
NW = NC * NS
PPW = N // NW
L = 16

_mesh = plsc.VectorSubcoreMesh(core_axis_name="c", subcore_axis_name="s")


@functools.partial(
    pl.kernel,
    mesh=_mesh,
    out_type=jax.ShapeDtypeStruct((2 * NW * KP,), jnp.float32),
    compiler_params=pltpu.CompilerParams(
        needs_layout_passes=False, use_tc_tiling_on_sc=False),
    scratch_types=[
        pltpu.VMEM((PPW,), jnp.float32),
        pltpu.VMEM((PPW,), jnp.float32),
        pltpu.VMEM((PPW,), jnp.float32),
        pltpu.VMEM((PPW,), jnp.float32),
        pltpu.VMEM((PPW,), jnp.float32),
        pltpu.VMEM((PPW,), jnp.float32),
        pltpu.VMEM((KP,), jnp.float32),
        pltpu.VMEM((KP,), jnp.float32),
        pltpu.VMEM((L,), jnp.float32),
    ],
)
def _hist_kernel(xin_hbm, xs_hbm, w_hbm, out_hbm,
                 a0, a1, a2, b0, b1, b2, hm, hl, wv):
    wid = lax.axis_index("s") * NC + lax.axis_index("c")
    base = wid * PPW

    pltpu.sync_copy(w_hbm, wv)
    pltpu.sync_copy(xin_hbm.at[pl.ds(base, PPW)], a0)
    pltpu.sync_copy(xin_hbm.at[pl.ds(N + base, PPW)], a1)
    pltpu.sync_copy(xin_hbm.at[pl.ds(2 * N + base, PPW)], a2)
    pltpu.sync_copy(xs_hbm.at[pl.ds(base, PPW)], b0)
    pltpu.sync_copy(xs_hbm.at[pl.ds(N + base, PPW)], b1)
    pltpu.sync_copy(xs_hbm.at[pl.ds(2 * N + base, PPW)], b2)

    def zero_body(i, carry):
        z = jnp.zeros((L,), jnp.float32)
        hm[pl.ds(i * L, L)] = z
        hl[pl.ds(i * L, L)] = z
        return carry

    lax.fori_loop(0, KP // L, zero_body, 0)

    wvec = wv[pl.ds(0, L)] * 0.0625
    w0 = wvec[0]
    w1 = wvec[1]
    w2 = wvec[2]
    ones = jnp.ones((L,), jnp.float32)

    def body(i, carry):
        o = i * L
        im = (a0[pl.ds(o, L)] * w0 + a1[pl.ds(o, L)] * w1
              + a2[pl.ds(o, L)] * w2).astype(jnp.int32)
        il = (b0[pl.ds(o, L)] * w0 + b1[pl.ds(o, L)] * w1
              + b2[pl.ds(o, L)] * w2).astype(jnp.int32)
        plsc.addupdate_scatter(hm, [im], ones)
        plsc.addupdate_scatter(hl, [il], ones)
        return carry

    lax.fori_loop(0, PPW // L, body, 0)

    pltpu.sync_copy(hm, out_hbm.at[pl.ds(wid * KP, KP)])
    pltpu.sync_copy(hl, out_hbm.at[pl.ds((NW + wid) * KP, KP)])


def _reduce_body(h_ref, tm_ref, tl_ref, o_ref):
    cm = jnp.sum(h_ref[:NW, :], axis=0, keepdims=True)
    cl = jnp.sum(h_ref[NW:, :], axis=0, keepdims=True)
    s = (jnp.dot(cm, tm_ref[...], precision=lax.Precision.HIGHEST,
                 preferred_element_type=jnp.float32)
         + jnp.dot(cl, tl_ref[...], precision=lax.Precision.HIGHEST,
                   preferred_element_type=jnp.float32))
    r = s * (1.0 / N)
    o_ref[...] = jnp.clip(jnp.round(r * 4.0) * 0.25, -32.0, 31.75)


@jax.jit
def kernel(x_in, x_s, feature_msb, feature_lsb, weights):
    xin = x_in.reshape(3 * N)
    xs = x_s.reshape(3 * N)
    wpad = jnp.pad(weights.reshape(3).astype(jnp.float32), (0, L - 3))

    hists = _hist_kernel(xin, xs, wpad).reshape(2 * NW, KP)

    tm = jnp.pad(feature_msb[::16, :, 0, 0], ((0, KP - KC), (0, 0)))
    tl = jnp.pad(feature_lsb[::16, :, 0, 0], ((0, KP - KC), (0, 0)))

    out = pl.pallas_call(
        _reduce_body,
        out_shape=jax.ShapeDtypeStruct((1, D), jnp.float32),
    )(hists, tm, tl)
    return out.reshape(1, D, 1, 1)
